# baseline (speedup 1.0000x reference)
import jax
import jax.numpy as jnp
from jax.experimental import pallas as pl
from jax.experimental.pallas import tpu as pltpu

N = 256
R = 8
NPANELS = N // R


def _panel_step(sref, lo, hi, kb):
    Bn = hi - lo
    rows = jax.lax.broadcasted_iota(jnp.int32, (1, N, 1), 1)
    cols = jax.lax.broadcasted_iota(jnp.int32, (1, 1, N), 2)
    ecol = jax.lax.broadcasted_iota(jnp.int32, (N, R), 0)
    eidx = jax.lax.broadcasted_iota(jnp.int32, (N, R), 1)
    i8c = jax.lax.broadcasted_iota(jnp.int32, (1, 1, R), 2)
    i8r = jax.lax.broadcasted_iota(jnp.int32, (1, R, 1), 1)

    A = sref[lo:hi]
    Rw = sref[lo:hi, pl.ds(kb, R), :]
    E = (ecol == eidx + kb).astype(jnp.float32)
    P = jax.lax.dot_general(A, E, (((2,), (0,)), ((), ())))
    S = jax.lax.dot_general(Rw, E, (((2,), (0,)), ((), ())))

    cs, rps = [], []
    for j in range(R):
        piv = S[:, j:j + 1, j:j + 1]
        rowj = Rw[:, j:j + 1, :]
        colj = P[:, :, j:j + 1]
        scol = S[:, :, j:j + 1]
        cmask = (cols == kb + j).astype(jnp.float32)
        c = jnp.where(rows > kb + j, colj / piv, 0.0)
        cpan = jnp.where(i8r > j, scol / piv, 0.0)
        rp = jnp.where(cols > kb + j, rowj, 0.0) + (piv - 1.0) * cmask
        rppan = (jnp.where(i8c > j, S[:, j:j + 1, :], 0.0)
                 + (piv - 1.0) * (i8c == j).astype(jnp.float32))
        Rw = Rw - cpan * rp
        P = P - c * rppan
        S = S - cpan * rppan
        cs.append(c)
        rps.append(rp)

    C = jnp.concatenate(cs, axis=2)
    Rm = jnp.concatenate(rps, axis=1)
    upd = jax.lax.dot_general(C, Rm, (((2,), (1,)), ((0,), (0,))))
    sref[lo:hi] = A - upd


def _masks_2x2():
    r = jax.lax.broadcasted_iota(jnp.int32, (N, N), 0)
    c = jax.lax.broadcasted_iota(jnp.int32, (N, N), 1)
    def m(i, j):
        return ((r == i) & (c == j)).astype(jnp.float32)
    return m


def _lu_kernel(x_ref, o_ref, s):
    s[0] = x_ref[0]
    s[1] = x_ref[1]
    s[2] = x_ref[2]
    s[3] = x_ref[5]
    s[4] = x_ref[6]

    m = _masks_2x2()

    def body(t, carry):
        @pl.when(t == 1)
        def _():
            b1, b2, b5, b6 = s[1], s[2], s[3], s[4]
            corr3 = ((b1[1:2, 1:2] + b2[2:3, 2:3]) * m(0, 0)
                     + b2[2:3, 3:4] * m(0, 1)
                     + b2[3:4, 2:3] * m(1, 0)
                     + b2[3:4, 3:4] * m(1, 1))
            corr7 = ((b5[1:2, 1:2] + b6[3:4, 3:4]) * m(0, 0)
                     + b6[3:4, 4:5] * m(0, 1)
                     + b6[4:5, 3:4] * m(1, 0)
                     + b6[4:5, 4:5] * m(1, 1))
            s[5] = x_ref[3] - corr3
            s[6] = x_ref[7] - corr7

        @pl.when(t == 2)
        def _():
            corr8 = (s[0, 1:2, 1:2] + s[5, 1:2, 1:2] + s[6, 1:2, 1:2]) * m(0, 0)
            s[7] = x_ref[8] - corr8

        @pl.when(t < NPANELS)
        def _():
            _panel_step(s, 0, 5, t * R)

        @pl.when((t >= 1) & (t < NPANELS + 1))
        def _():
            _panel_step(s, 5, 7, (t - 1) * R)

        @pl.when((t >= 2) & (t < NPANELS + 2))
        def _():
            _panel_step(s, 7, 8, (t - 2) * R)

        return carry

    jax.lax.fori_loop(0, NPANELS + 2, body, 0)

    o_ref[0] = s[0]
    o_ref[1] = s[1]
    o_ref[2] = s[2]
    o_ref[3] = s[5]
    o_ref[4] = x_ref[4]
    o_ref[5] = s[3]
    o_ref[6] = s[4]
    o_ref[7] = s[6]
    o_ref[8] = s[7]


def kernel(input):
    return pl.pallas_call(
        _lu_kernel,
        out_shape=jax.ShapeDtypeStruct((9, N, N), jnp.float32),
        scratch_shapes=[pltpu.VMEM((8, N, N), jnp.float32)],
    )(input)
